# baseline (speedup 1.0000x reference)
import functools

import jax
import jax.numpy as jnp
from jax.experimental import pallas as pl
from jax.experimental.pallas import tpu as pltpu

N_NODES = 10000
E_EDGES = 320000
N_CH = 4


def _mm_body(x_ref, w_ref, o_ref):
    o_ref[:] = jnp.dot(x_ref[:], w_ref[:], preferred_element_type=jnp.float32)


def _proj2_body(a_ref, b_ref, wa_ref, wb_ref, o_ref):
    o_ref[:] = (jnp.dot(a_ref[:], wa_ref[:], preferred_element_type=jnp.float32)
                + jnp.dot(b_ref[:], wb_ref[:], preferred_element_type=jnp.float32))


def _mid_body(d, p_ref, b_ref, w_ref, o_ref):
    h = jax.nn.relu(p_ref[0] + p_ref[1] + b_ref[:])[:, :d]
    o_ref[:] = jnp.dot(h, w_ref[:], preferred_element_type=jnp.float32)


def _final_body(d, p_ref, b3_ref, l1w_ref, l1b_ref, l2w_ref, l2b_ref,
                l3w_ref, l3b_ref, o_ref):
    h = jax.nn.relu(p_ref[0] + p_ref[1] + b3_ref[:])[:, :d]
    h = jax.nn.relu(jnp.dot(h, l1w_ref[:].T, preferred_element_type=jnp.float32) + l1b_ref[:])
    h = jax.nn.relu(jnp.dot(h, l2w_ref[:].T, preferred_element_type=jnp.float32) + l2b_ref[:])
    z = jnp.sum(h * l3w_ref[:], axis=1, keepdims=True) + l3b_ref[0]
    o_ref[:] = jax.nn.sigmoid(z)


def _tc_call(body, out_shape, *args):
    return pl.pallas_call(
        body,
        out_shape=jax.ShapeDtypeStruct(out_shape, jnp.float32),
    )(*args)


def _edge_pass(Y, src, dst, vals, d):
    out = jnp.zeros((N_NODES, d), dtype=jnp.float32)
    for c in range(N_CH):
        msg = Y[:, c, :][src] * vals[:, c][:, None]
        out = out + jax.ops.segment_sum(msg, dst, num_segments=N_NODES)
    return out


def kernel(one_hot, features, gemme_features, a_res_indices, a_res_values,
           W1, b1, W2, b2, W3, b3, l1w, l1b, l2w, l2b, l3w, l3b):
    src = a_res_indices[0].astype(jnp.int32)
    dst = a_res_indices[1].astype(jnp.int32)
    vals = a_res_values

    d1, d2, d3 = W1.shape[2], W2.shape[2], W3.shape[2]
    W1s = jnp.transpose(W1, (1, 0, 2)).reshape(W1.shape[1], N_CH * d1)
    W2s = jnp.transpose(W2, (1, 0, 2)).reshape(W2.shape[1], N_CH * d2)
    W3s = jnp.transpose(W3, (1, 0, 2)).reshape(W3.shape[1], N_CH * d3)

    r = one_hot.shape[1]
    Y1 = _tc_call(_proj2_body, (N_NODES, N_CH * d1),
                  one_hot, features, W1s[:r], W1s[r:])
    agg1 = _edge_pass(Y1.reshape(N_NODES, N_CH, d1), src, dst, vals, d1)
    p1 = jnp.stack([agg1, jnp.zeros_like(agg1)])

    Y2 = _tc_call(functools.partial(_mid_body, d1), (N_NODES, N_CH * d2),
                  p1, b1, W2s)
    agg2 = _edge_pass(Y2.reshape(N_NODES, N_CH, d2), src, dst, vals, d2)
    p2 = jnp.stack([agg2, jnp.zeros_like(agg2)])

    Y3 = _tc_call(functools.partial(_mid_body, d2), (N_NODES, N_CH * d3),
                  p2, b2, W3s)
    agg3 = _edge_pass(Y3.reshape(N_NODES, N_CH, d3), src, dst, vals, d3)
    p3 = jnp.stack([agg3, jnp.zeros_like(agg3)])

    return _tc_call(functools.partial(_final_body, d3), (N_NODES, 1),
                    p3, b3, l1w, l1b, l2w, l2b, l3w, l3b)


# speedup vs baseline: 13.0123x; 13.0123x over previous
import functools

import jax
import jax.numpy as jnp
from jax import lax
from jax.experimental import pallas as pl
from jax.experimental.pallas import tpu as pltpu
from jax.experimental.pallas import tpu_sc as plsc

N_NODES = 10000
E_EDGES = 320000
N_CH = 4

NC = 2
NS = 16
NW = NC * NS
B = 80
NBT = E_EDGES // B
NB = NBT // NW
NPT = N_NODES // NS


def _proj2_body(a_ref, b_ref, wa_ref, wb_ref, o_ref):
    o_ref[:] = (jnp.dot(a_ref[:], wa_ref[:], preferred_element_type=jnp.float32)
                + jnp.dot(b_ref[:], wb_ref[:], preferred_element_type=jnp.float32))


def _mid_body(d, p_ref, b_ref, w_ref, o_ref):
    h = jax.nn.relu(p_ref[0] + p_ref[1] + b_ref[:])[:, :d]
    o_ref[:] = jnp.dot(h, w_ref[:], preferred_element_type=jnp.float32)


def _final_body(d, p_ref, b3_ref, l1w_ref, l1b_ref, l2w_ref, l2b_ref,
                l3w_ref, l3b_ref, o_ref):
    h = jax.nn.relu(p_ref[0] + p_ref[1] + b3_ref[:])[:, :d]
    h = jax.nn.relu(jnp.dot(h, l1w_ref[:].T, preferred_element_type=jnp.float32) + l1b_ref[:])
    h = jax.nn.relu(jnp.dot(h, l2w_ref[:].T, preferred_element_type=jnp.float32) + l2b_ref[:])
    z = jnp.sum(h * l3w_ref[:], axis=1, keepdims=True) + l3b_ref[0]
    o_ref[:] = jax.nn.sigmoid(z)


def _tc_call(body, out_shape, *args):
    return pl.pallas_call(
        body,
        out_shape=jax.ShapeDtypeStruct(out_shape, jnp.float32),
    )(*args)


def _sc_edge_kernel(dp):
    R = N_CH * dp
    nv = dp // 16

    def body(y_hbm, src_hbm, dst_hbm, vals_hbm, out_hbm,
             src_v, dst_v, vals_v, rows_v, msgs_v, zbuf_v, acc_sh, sem):
        cid = lax.axis_index("c")
        sid = lax.axis_index("s")
        wid = cid * NS + sid

        def zrow(r, carry):
            for k in range(nv):
                zbuf_v[r, pl.ds(16 * k, 16)] = jnp.zeros((16,), jnp.float32)
            return carry
        lax.fori_loop(0, NPT, zrow, 0)
        pltpu.sync_copy(zbuf_v, acc_sh.at[pl.ds(sid * NPT, NPT)])

        pltpu.sync_copy(src_hbm.at[wid], src_v)
        pltpu.sync_copy(dst_hbm.at[wid], dst_v)
        pltpu.sync_copy(vals_hbm.at[wid], vals_v)
        plsc.subcore_barrier()

        def batch(i, carry):
            pltpu.async_copy(y_hbm.at[src_v.at[i]], rows_v, sem).wait()

            def group(g, c2):
                vv = vals_v[pl.ds(i * (N_CH * B) + g * 16, 16)]
                for le in range(4):
                    e = g * 4 + le
                    accs = [None] * nv
                    for c in range(N_CH):
                        s = vv.at[jnp.full((16,), le * N_CH + c, jnp.int32)].get(
                            mode="promise_in_bounds")
                        for k in range(nv):
                            t = s * rows_v[e, pl.ds(c * dp + 16 * k, 16)]
                            accs[k] = t if accs[k] is None else accs[k] + t
                    for k in range(nv):
                        msgs_v[e, pl.ds(16 * k, 16)] = accs[k]
                return c2
            lax.fori_loop(0, B // 4, group, 0)

            pltpu.sync_copy(msgs_v, acc_sh.at[dst_v.at[i]], add=True)
            return carry
        lax.fori_loop(0, NB, batch, 0)

        plsc.subcore_barrier()
        pltpu.sync_copy(acc_sh.at[pl.ds(sid * NPT, NPT)], out_hbm.at[cid, sid])

    mesh = plsc.VectorSubcoreMesh(core_axis_name="c", subcore_axis_name="s",
                                  num_cores=NC, num_subcores=NS)
    return pl.kernel(
        body,
        out_type=jax.ShapeDtypeStruct((NC, NS, NPT, dp), jnp.float32),
        mesh=mesh,
        compiler_params=pltpu.CompilerParams(use_tc_tiling_on_sc=False),
        scratch_types=[
            pltpu.VMEM((NB, B), jnp.int32),
            pltpu.VMEM((NB, B), jnp.int32),
            pltpu.VMEM((NB * N_CH * B,), jnp.float32),
            pltpu.VMEM((B, R), jnp.float32),
            pltpu.VMEM((B, dp), jnp.float32),
            pltpu.VMEM((NPT, dp), jnp.float32),
            pltpu.VMEM_SHARED((N_NODES, dp), jnp.float32),
            pltpu.SemaphoreType.DMA,
        ],
    )


def _stack_pad(W, dp):
    d_in, d = W.shape[1], W.shape[2]
    Wt = jnp.transpose(W, (1, 0, 2))
    Wt = jnp.pad(Wt, ((0, 0), (0, 0), (0, dp - d)))
    return Wt.reshape(d_in, N_CH * dp)


def kernel(one_hot, features, gemme_features, a_res_indices, a_res_values,
           W1, b1, W2, b2, W3, b3, l1w, l1b, l2w, l2b, l3w, l3b):
    src = a_res_indices[0].astype(jnp.int32).reshape(NW, NB, B)
    dst = a_res_indices[1].astype(jnp.int32).reshape(NW, NB, B)
    vals = a_res_values.reshape(NW, NB * N_CH * B)

    d1, d2, d3 = W1.shape[2], W2.shape[2], W3.shape[2]
    dp1, dp2, dp3 = 32, 16, 16
    W1s = _stack_pad(W1, dp1)
    W2s = _stack_pad(W2, dp2)
    W3s = _stack_pad(W3, dp3)
    b1p = jnp.pad(b1, (0, dp1 - d1))
    b2p = jnp.pad(b2, (0, dp2 - d2))
    b3p = jnp.pad(b3, (0, dp3 - d3))

    edge1 = _sc_edge_kernel(dp1)
    edge23 = _sc_edge_kernel(dp2)

    r = one_hot.shape[1]
    Y1 = _tc_call(_proj2_body, (N_NODES, N_CH * dp1),
                  one_hot, features, W1s[:r], W1s[r:])
    p1 = edge1(Y1, src, dst, vals).reshape(NC, N_NODES, dp1)

    Y2 = _tc_call(functools.partial(_mid_body, d1), (N_NODES, N_CH * dp2),
                  p1, b1p, W2s)
    p2 = edge23(Y2, src, dst, vals).reshape(NC, N_NODES, dp2)

    Y3 = _tc_call(functools.partial(_mid_body, d2), (N_NODES, N_CH * dp3),
                  p2, b2p, W3s)
    p3 = edge23(Y3, src, dst, vals).reshape(NC, N_NODES, dp3)

    return _tc_call(functools.partial(_final_body, d3), (N_NODES, 1),
                    p3, b3p, l1w, l1b, l2w, l2b, l3w, l3b)


# speedup vs baseline: 21.1216x; 1.6232x over previous
import functools

import jax
import jax.numpy as jnp
from jax import lax
from jax.experimental import pallas as pl
from jax.experimental.pallas import tpu as pltpu
from jax.experimental.pallas import tpu_sc as plsc

N_NODES = 10000
E_EDGES = 320000
N_CH = 4

NC = 2
NS = 16
NW = NC * NS
B = 100
NB = E_EDGES // (B * NW)
NBUF = 4
NPT = N_NODES // NS


def _proj2_body(a_ref, b_ref, wa_ref, wb_ref, o_ref):
    o_ref[:] = (jnp.dot(a_ref[:], wa_ref[:], preferred_element_type=jnp.float32)
                + jnp.dot(b_ref[:], wb_ref[:], preferred_element_type=jnp.float32))


def _mid_body(d, p_ref, b_ref, w_ref, o_ref):
    h = jax.nn.relu(p_ref[0] + p_ref[1] + b_ref[:])[:, :d]
    o_ref[:] = jnp.dot(h, w_ref[:], preferred_element_type=jnp.float32)


def _final_body(d, p_ref, b3_ref, l1w_ref, l1b_ref, l2w_ref, l2b_ref,
                l3w_ref, l3b_ref, o_ref):
    h = jax.nn.relu(p_ref[0] + p_ref[1] + b3_ref[:])[:, :d]
    h = jax.nn.relu(jnp.dot(h, l1w_ref[:].T, preferred_element_type=jnp.float32) + l1b_ref[:])
    h = jax.nn.relu(jnp.dot(h, l2w_ref[:].T, preferred_element_type=jnp.float32) + l2b_ref[:])
    z = jnp.sum(h * l3w_ref[:], axis=1, keepdims=True) + l3b_ref[0]
    o_ref[:] = jax.nn.sigmoid(z)


def _tc_call(body, out_shape, *args):
    return pl.pallas_call(
        body,
        out_shape=jax.ShapeDtypeStruct(out_shape, jnp.float32),
    )(*args)


def _sc_edge_kernel(dp):
    R = N_CH * dp
    nv = dp // 16

    def body(y_hbm, src_hbm, dst_hbm, vals_hbm, out_hbm,
             src_v, dst_v, vals_v, rows_v, msgs_v, zbuf_v, acc_sh,
             gs0, gs1, gs2, gs3, vs0, vs1, vs2, vs3, ss0, ss1):
        gsems = (gs0, gs1, gs2, gs3)
        vsems = (vs0, vs1, vs2, vs3)
        ssems = (ss0, ss1)
        cid = lax.axis_index("c")
        sid = lax.axis_index("s")
        wid = cid * NS + sid

        def zrow(r, carry):
            for k in range(nv):
                zbuf_v[r, pl.ds(16 * k, 16)] = jnp.zeros((16,), jnp.float32)
            return carry
        lax.fori_loop(0, NPT // 5, zrow, 0)
        for j in range(5):
            pltpu.sync_copy(zbuf_v,
                            acc_sh.at[pl.ds(sid * NPT + j * (NPT // 5), NPT // 5)])

        pltpu.sync_copy(src_hbm.at[wid], src_v)
        pltpu.sync_copy(dst_hbm.at[wid], dst_v)
        plsc.subcore_barrier()

        def start_gather(i, b):
            pltpu.async_copy(y_hbm.at[src_v.at[i]], rows_v.at[b], gsems[b])
            pltpu.async_copy(vals_hbm.at[wid, pl.ds(i * (N_CH * B), N_CH * B)],
                             vals_v.at[b], vsems[b])

        def wait_gather(i, b):
            pltpu.make_async_copy(y_hbm.at[src_v.at[i]], rows_v.at[b],
                                  gsems[b]).wait()
            pltpu.make_async_copy(vals_hbm.at[wid, pl.ds(i * (N_CH * B), N_CH * B)],
                                  vals_v.at[b], vsems[b]).wait()

        def start_scatter(i, m):
            pltpu.async_copy(msgs_v.at[m], acc_sh.at[dst_v.at[i]], ssems[m],
                             add=True)

        def wait_scatter(i, m):
            pltpu.make_async_copy(msgs_v.at[m], acc_sh.at[dst_v.at[i]],
                                  ssems[m]).wait()

        def compute(i, b, m):
            def group(g, c2):
                vv = vals_v[b, pl.ds(g * 16, 16)]
                for le in range(4):
                    e = g * 4 + le
                    accs = [None] * nv
                    for c in range(N_CH):
                        s = vv.at[jnp.full((16,), le * N_CH + c, jnp.int32)].get(
                            mode="promise_in_bounds")
                        for k in range(nv):
                            t = s * rows_v[b, e, pl.ds(c * dp + 16 * k, 16)]
                            accs[k] = t if accs[k] is None else accs[k] + t
                    for k in range(nv):
                        msgs_v[m, e, pl.ds(16 * k, 16)] = accs[k]
                return c2
            lax.fori_loop(0, B // 4, group, 0)

        for b in range(NBUF - 1):
            start_gather(b, b)

        def quad(q, carry):
            i0 = q * NBUF
            for b in range(NBUF):
                i = i0 + b

                @pl.when(i + NBUF - 1 < NB)
                def _():
                    start_gather(i + NBUF - 1, (b + NBUF - 1) % NBUF)
                wait_gather(i, b)
                m = b % 2

                @pl.when(i >= 2)
                def _():
                    wait_scatter(i - 2, m)
                compute(i, b, m)
                start_scatter(i, m)
            return carry
        lax.fori_loop(0, NB // NBUF, quad, 0)
        wait_scatter(NB - 2, 0)
        wait_scatter(NB - 1, 1)

        plsc.subcore_barrier()
        pltpu.sync_copy(acc_sh.at[pl.ds(sid * NPT, NPT)], out_hbm.at[cid, sid])

    mesh = plsc.VectorSubcoreMesh(core_axis_name="c", subcore_axis_name="s",
                                  num_cores=NC, num_subcores=NS)
    return pl.kernel(
        body,
        out_type=jax.ShapeDtypeStruct((NC, NS, NPT, dp), jnp.float32),
        mesh=mesh,
        compiler_params=pltpu.CompilerParams(use_tc_tiling_on_sc=False),
        scratch_types=[
            pltpu.VMEM((NB, B), jnp.int32),
            pltpu.VMEM((NB, B), jnp.int32),
            pltpu.VMEM((NBUF, N_CH * B), jnp.float32),
            pltpu.VMEM((NBUF, B, R), jnp.float32),
            pltpu.VMEM((2, B, dp), jnp.float32),
            pltpu.VMEM((NPT // 5, dp), jnp.float32),
            pltpu.VMEM_SHARED((N_NODES, dp), jnp.float32),
        ] + [pltpu.SemaphoreType.DMA] * 10,
    )


def _stack_pad(W, dp):
    d_in, d = W.shape[1], W.shape[2]
    Wt = jnp.transpose(W, (1, 0, 2))
    Wt = jnp.pad(Wt, ((0, 0), (0, 0), (0, dp - d)))
    return Wt.reshape(d_in, N_CH * dp)


def kernel(one_hot, features, gemme_features, a_res_indices, a_res_values,
           W1, b1, W2, b2, W3, b3, l1w, l1b, l2w, l2b, l3w, l3b):
    src = a_res_indices[0].astype(jnp.int32).reshape(NW, NB, B)
    dst = a_res_indices[1].astype(jnp.int32).reshape(NW, NB, B)
    vals = a_res_values.reshape(NW, NB * N_CH * B)

    d1, d2, d3 = W1.shape[2], W2.shape[2], W3.shape[2]
    dp1, dp2, dp3 = 32, 16, 16
    W1s = _stack_pad(W1, dp1)
    W2s = _stack_pad(W2, dp2)
    W3s = _stack_pad(W3, dp3)
    b1p = jnp.pad(b1, (0, dp1 - d1))
    b2p = jnp.pad(b2, (0, dp2 - d2))
    b3p = jnp.pad(b3, (0, dp3 - d3))

    edge1 = _sc_edge_kernel(dp1)
    edge23 = _sc_edge_kernel(dp2)

    r = one_hot.shape[1]
    Y1 = _tc_call(_proj2_body, (N_NODES, N_CH * dp1),
                  one_hot, features, W1s[:r], W1s[r:])
    p1 = edge1(Y1, src, dst, vals).reshape(NC, N_NODES, dp1)

    Y2 = _tc_call(functools.partial(_mid_body, d1), (N_NODES, N_CH * dp2),
                  p1, b1p, W2s)
    p2 = edge23(Y2, src, dst, vals).reshape(NC, N_NODES, dp2)

    Y3 = _tc_call(functools.partial(_mid_body, d2), (N_NODES, N_CH * dp3),
                  p2, b2p, W3s)
    p3 = edge23(Y3, src, dst, vals).reshape(NC, N_NODES, dp3)

    return _tc_call(functools.partial(_final_body, d3), (N_NODES, 1),
                    p3, b3p, l1w, l1b, l2w, l2b, l3w, l3b)
